# baseline (speedup 1.0000x reference)
import functools

import jax
import jax.numpy as jnp
from jax import lax
from jax.experimental import pallas as pl
from jax.experimental.pallas import tpu as pltpu
from jax.experimental.pallas import tpu_sc as plsc

_NC = 2
_NS = 16
_NW = _NC * _NS
_CH = 128
_NBUF = 5


def _deg_hist_kernel(npad, ept):
    mesh = plsc.VectorSubcoreMesh(core_axis_name="c", subcore_axis_name="s")

    @functools.partial(
        pl.kernel,
        out_type=jax.ShapeDtypeStruct((_NW, npad), jnp.float32),
        mesh=mesh,
        scratch_types=[
            pltpu.VMEM((ept,), jnp.int32),
            pltpu.VMEM((npad,), jnp.float32),
        ],
        compiler_params=pltpu.CompilerParams(needs_layout_passes=False),
    )
    def deg_hist(dst_hbm, zrow_hbm, hist_hbm, dst_v, hist_v):
        c = lax.axis_index("c")
        s = lax.axis_index("s")
        w = s * _NC + c
        pltpu.sync_copy(dst_hbm.at[w], dst_v)
        pltpu.sync_copy(zrow_hbm, hist_v)
        ones = jnp.full((16,), 1.0, jnp.float32)

        def body(i, carry):
            for u in range(8):
                idx = dst_v[pl.ds((i * 8 + u) * 16, 16)]
                plsc.addupdate_scatter(hist_v, [idx], ones)
            return carry

        lax.fori_loop(0, ept // 128, body, 0)
        pltpu.sync_copy(hist_v, hist_hbm.at[w])

    return deg_hist


_NSTAGE = 8


def _edge_scatter_kernel(npad, nh, cpt):
    mesh = plsc.VectorSubcoreMesh(core_axis_name="c", subcore_axis_name="s")
    rpt = npad // _NS
    spc = cpt // _NSTAGE
    rounds = spc // _NBUF

    @functools.partial(
        pl.kernel,
        out_type=jax.ShapeDtypeStruct((_NC, npad, nh), jnp.float32),
        mesh=mesh,
        scratch_types=[
            pltpu.VMEM((spc, _CH), jnp.int32),
            pltpu.VMEM((spc, _CH), jnp.int32),
            pltpu.VMEM((_CH, nh), jnp.float32),
            pltpu.VMEM((_CH, nh), jnp.float32),
            pltpu.VMEM((_CH, nh), jnp.float32),
            pltpu.VMEM((_CH, nh), jnp.float32),
            pltpu.VMEM((_CH, nh), jnp.float32),
            pltpu.VMEM_SHARED((npad, nh), jnp.float32),
            pltpu.VMEM_SHARED((npad, nh), jnp.float32),
            pltpu.SemaphoreType.DMA,
            pltpu.SemaphoreType.DMA,
            pltpu.SemaphoreType.DMA,
            pltpu.SemaphoreType.DMA,
            pltpu.SemaphoreType.DMA,
            pltpu.SemaphoreType.DMA,
            pltpu.SemaphoreType.DMA,
            pltpu.SemaphoreType.DMA,
            pltpu.SemaphoreType.DMA,
            pltpu.SemaphoreType.DMA,
        ],
        compiler_params=pltpu.CompilerParams(use_tc_tiling_on_sc=False),
    )
    def edge_scatter(g_hbm, src_hbm, dst_hbm, zero_hbm, part_hbm,
                     src_v, dst_v, b0, b1, b2, b3, b4, table, acc,
                     gs0, gs1, gs2, gs3, gs4, ss0, ss1, ss2, ss3, ss4):
        bufs = [b0, b1, b2, b3, b4]
        gsems = [gs0, gs1, gs2, gs3, gs4]
        ssems = [ss0, ss1, ss2, ss3, ss4]
        c = lax.axis_index("c")
        s = lax.axis_index("s")
        pltpu.sync_copy(g_hbm.at[c].at[pl.ds(s * rpt, rpt)],
                        table.at[pl.ds(s * rpt, rpt)])
        pltpu.sync_copy(zero_hbm.at[pl.ds(s * rpt, rpt)],
                        acc.at[pl.ds(s * rpt, rpt)])
        plsc.subcore_barrier()

        for q in range(_NSTAGE):
            pltpu.sync_copy(src_hbm.at[s].at[pl.ds(q * spc, spc)], src_v)
            pltpu.sync_copy(dst_hbm.at[s].at[pl.ds(q * spc, spc)], dst_v)

            def body(t, carry):
                for b in range(_NBUF):
                    j = t * _NBUF + b
                    pltpu.async_copy(table.at[src_v.at[j]], bufs[b],
                                     gsems[b])
                for b in range(_NBUF):
                    j = t * _NBUF + b
                    pltpu.make_async_copy(table.at[src_v.at[j]], bufs[b],
                                          gsems[b]).wait()
                    pltpu.async_copy(bufs[b], acc.at[dst_v.at[j]],
                                     ssems[b], add=True)
                for b in range(_NBUF):
                    j = t * _NBUF + b
                    pltpu.make_async_copy(bufs[b], acc.at[dst_v.at[j]],
                                          ssems[b]).wait()
                return carry

            lax.fori_loop(0, rounds, body, 0)

        plsc.subcore_barrier()
        pltpu.sync_copy(acc.at[pl.ds(s * rpt, rpt)],
                        part_hbm.at[c].at[pl.ds(s * rpt, rpt)])

    return edge_scatter


def kernel(x, edge_index, W, b):
    n, nfeat = x.shape
    nhid = W.shape[1]
    nh = nhid // 2
    e = edge_index.shape[1]

    npad = ((n + 1 + 511) // 512) * 512

    src = edge_index[0]
    dst = edge_index[1]

    cpt = -(-e // (_NS * _CH))
    cpt = -(-cpt // (_NBUF * _NSTAGE)) * (_NBUF * _NSTAGE)
    epad = _NS * cpt * _CH
    fill = jnp.full((epad - e,), n, jnp.int32)
    src_p = jnp.concatenate([src, fill])
    dst_p = jnp.concatenate([dst, fill])
    src_sc = src_p.reshape(_NS, cpt, _CH)
    dst_sc = dst_p.reshape(_NS, cpt, _CH)

    ept32 = epad // _NW
    assert ept32 % 16 == 0
    dst_flat = dst_p.reshape(_NW, ept32)

    zrow = jnp.zeros((npad,), jnp.float32)
    zhalf = jnp.zeros((npad, nh), jnp.float32)

    hist = _deg_hist_kernel(npad, ept32)(dst_flat, zrow)

    rb = 512

    def g_body(x_ref, w_ref, hist_ref, g_ref, dinv_ref):
        deg = jnp.sum(hist_ref[...], axis=0) + 1.0
        dinv = lax.rsqrt(deg)
        h = jnp.dot(x_ref[...], w_ref[...],
                    preferred_element_type=jnp.float32)
        hd = h * dinv[:, None]
        g_ref[0] = hd[:, :nh]
        g_ref[1] = hd[:, nh:]
        dinv_ref[...] = dinv[:, None]

    g2, dinv2 = pl.pallas_call(
        g_body,
        grid=(npad // rb,),
        in_specs=[
            pl.BlockSpec((rb, nfeat), lambda i: (i, 0)),
            pl.BlockSpec((nfeat, nhid), lambda i: (0, 0)),
            pl.BlockSpec((_NW, rb), lambda i: (0, i)),
        ],
        out_specs=[
            pl.BlockSpec((_NC, rb, nh), lambda i: (0, i, 0)),
            pl.BlockSpec((rb, 1), lambda i: (i, 0)),
        ],
        out_shape=[
            jax.ShapeDtypeStruct((_NC, npad, nh), jnp.float32),
            jax.ShapeDtypeStruct((npad, 1), jnp.float32),
        ],
    )(x, W, hist)

    part = _edge_scatter_kernel(npad, nh, cpt)(g2, src_sc, dst_sc, zhalf)

    fb = 400
    assert n % fb == 0

    def fin_body(p_ref, g_ref, dinv_ref, b_ref, o_ref):
        dinv = dinv_ref[...]
        a0 = (p_ref[0] + g_ref[0]) * dinv
        a1 = (p_ref[1] + g_ref[1]) * dinv
        acc = jnp.concatenate([a0, a1], axis=1)
        o_ref[...] = jnp.maximum(acc + b_ref[...], 0.0)

    out = pl.pallas_call(
        fin_body,
        grid=(n // fb,),
        in_specs=[
            pl.BlockSpec((_NC, fb, nh), lambda i: (0, i, 0)),
            pl.BlockSpec((_NC, fb, nh), lambda i: (0, i, 0)),
            pl.BlockSpec((fb, 1), lambda i: (i, 0)),
            pl.BlockSpec((1, nhid), lambda i: (0, 0)),
        ],
        out_specs=pl.BlockSpec((fb, nhid), lambda i: (i, 0)),
        out_shape=jax.ShapeDtypeStruct((n, nhid), jnp.float32),
    )(part, g2, dinv2, b.reshape(1, nhid))

    return out


# speedup vs baseline: 1.1375x; 1.1375x over previous
import functools

import jax
import jax.numpy as jnp
from jax import lax
from jax.experimental import pallas as pl
from jax.experimental.pallas import tpu as pltpu
from jax.experimental.pallas import tpu_sc as plsc

_NC = 2
_NS = 16
_NW = _NC * _NS
_CH = 128
_NBUF = 4


def _deg_hist_kernel(npad, ept):
    mesh = plsc.VectorSubcoreMesh(core_axis_name="c", subcore_axis_name="s")

    @functools.partial(
        pl.kernel,
        out_type=jax.ShapeDtypeStruct((_NW, npad), jnp.float32),
        mesh=mesh,
        scratch_types=[
            pltpu.VMEM((ept,), jnp.int32),
            pltpu.VMEM((npad,), jnp.float32),
        ],
        compiler_params=pltpu.CompilerParams(needs_layout_passes=False),
    )
    def deg_hist(dst_hbm, zrow_hbm, hist_hbm, dst_v, hist_v):
        c = lax.axis_index("c")
        s = lax.axis_index("s")
        w = s * _NC + c
        pltpu.sync_copy(dst_hbm.at[w], dst_v)
        pltpu.sync_copy(zrow_hbm, hist_v)
        ones = jnp.full((16,), 1.0, jnp.float32)

        def body(i, carry):
            for u in range(8):
                idx = dst_v[pl.ds((i * 8 + u) * 16, 16)]
                plsc.addupdate_scatter(hist_v, [idx], ones)
            return carry

        lax.fori_loop(0, ept // 128, body, 0)
        pltpu.sync_copy(hist_v, hist_hbm.at[w])

    return deg_hist


_NSTAGE = 4


def _edge_scatter_kernel(npad, nh, cpt):
    mesh = plsc.VectorSubcoreMesh(core_axis_name="c", subcore_axis_name="s")
    rpt = npad // _NS
    spc = cpt // _NSTAGE
    rounds = spc // _NBUF

    @functools.partial(
        pl.kernel,
        out_type=jax.ShapeDtypeStruct((_NC, npad, nh), jnp.float32),
        mesh=mesh,
        scratch_types=[
            pltpu.VMEM((spc, _CH), jnp.int32),
            pltpu.VMEM((spc, _CH), jnp.int32),
            pltpu.VMEM((_CH, nh), jnp.float32),
            pltpu.VMEM((_CH, nh), jnp.float32),
            pltpu.VMEM((_CH, nh), jnp.float32),
            pltpu.VMEM((_CH, nh), jnp.float32),
            pltpu.VMEM_SHARED((npad, nh), jnp.float32),
            pltpu.VMEM_SHARED((npad, nh), jnp.float32),
            pltpu.SemaphoreType.DMA,
            pltpu.SemaphoreType.DMA,
            pltpu.SemaphoreType.DMA,
            pltpu.SemaphoreType.DMA,
            pltpu.SemaphoreType.DMA,
            pltpu.SemaphoreType.DMA,
            pltpu.SemaphoreType.DMA,
            pltpu.SemaphoreType.DMA,
        ],
        compiler_params=pltpu.CompilerParams(use_tc_tiling_on_sc=False),
    )
    def edge_scatter(g_hbm, src_hbm, dst_hbm, zero_hbm, part_hbm,
                     src_v, dst_v, b0, b1, b2, b3, table, acc,
                     gs0, gs1, gs2, gs3, ss0, ss1, ss2, ss3):
        bufs = [b0, b1, b2, b3]
        gsems = [gs0, gs1, gs2, gs3]
        ssems = [ss0, ss1, ss2, ss3]
        c = lax.axis_index("c")
        s = lax.axis_index("s")
        pltpu.sync_copy(g_hbm.at[c].at[pl.ds(s * rpt, rpt)],
                        table.at[pl.ds(s * rpt, rpt)])
        pltpu.sync_copy(zero_hbm.at[pl.ds(s * rpt, rpt)],
                        acc.at[pl.ds(s * rpt, rpt)])
        plsc.subcore_barrier()

        for q in range(_NSTAGE):
            pltpu.sync_copy(src_hbm.at[s].at[pl.ds(q * spc, spc)], src_v)
            pltpu.sync_copy(dst_hbm.at[s].at[pl.ds(q * spc, spc)], dst_v)

            def body(t, carry):
                for b in range(_NBUF):
                    j = t * _NBUF + b
                    pltpu.async_copy(table.at[src_v.at[j]], bufs[b],
                                     gsems[b])
                for b in range(_NBUF):
                    j = t * _NBUF + b
                    pltpu.make_async_copy(table.at[src_v.at[j]], bufs[b],
                                          gsems[b]).wait()
                    pltpu.async_copy(bufs[b], acc.at[dst_v.at[j]],
                                     ssems[b], add=True)
                for b in range(_NBUF):
                    j = t * _NBUF + b
                    pltpu.make_async_copy(bufs[b], acc.at[dst_v.at[j]],
                                          ssems[b]).wait()
                return carry

            lax.fori_loop(0, rounds, body, 0)

        plsc.subcore_barrier()
        pltpu.sync_copy(acc.at[pl.ds(s * rpt, rpt)],
                        part_hbm.at[c].at[pl.ds(s * rpt, rpt)])

    return edge_scatter


def kernel(x, edge_index, W, b):
    n, nfeat = x.shape
    nhid = W.shape[1]
    nh = nhid // 2
    e = edge_index.shape[1]

    npad = ((n + 1 + 511) // 512) * 512

    src = edge_index[0]
    dst = edge_index[1]

    cpt = -(-e // (_NS * _CH))
    cpt = -(-cpt // (_NBUF * _NSTAGE)) * (_NBUF * _NSTAGE)
    epad = _NS * cpt * _CH
    fill = jnp.full((epad - e,), n, jnp.int32)
    src_p = jnp.concatenate([src, fill])
    dst_p = jnp.concatenate([dst, fill])
    src_sc = src_p.reshape(_NS, cpt, _CH)
    dst_sc = dst_p.reshape(_NS, cpt, _CH)

    ept32 = epad // _NW
    assert ept32 % 16 == 0
    dst_flat = dst_p.reshape(_NW, ept32)

    zrow = jnp.zeros((npad,), jnp.float32)
    zhalf = jnp.zeros((npad, nh), jnp.float32)

    hist = _deg_hist_kernel(npad, ept32)(dst_flat, zrow)

    rb = 512

    def g_body(x_ref, w_ref, hist_ref, g_ref, dinv_ref):
        deg = jnp.sum(hist_ref[...], axis=0) + 1.0
        dinv = lax.rsqrt(deg)
        h = jnp.dot(x_ref[...], w_ref[...],
                    preferred_element_type=jnp.float32)
        hd = h * dinv[:, None]
        g_ref[0] = hd[:, :nh]
        g_ref[1] = hd[:, nh:]
        dinv_ref[...] = dinv[:, None]

    g2, dinv2 = pl.pallas_call(
        g_body,
        grid=(npad // rb,),
        in_specs=[
            pl.BlockSpec((rb, nfeat), lambda i: (i, 0)),
            pl.BlockSpec((nfeat, nhid), lambda i: (0, 0)),
            pl.BlockSpec((_NW, rb), lambda i: (0, i)),
        ],
        out_specs=[
            pl.BlockSpec((_NC, rb, nh), lambda i: (0, i, 0)),
            pl.BlockSpec((rb, 1), lambda i: (i, 0)),
        ],
        out_shape=[
            jax.ShapeDtypeStruct((_NC, npad, nh), jnp.float32),
            jax.ShapeDtypeStruct((npad, 1), jnp.float32),
        ],
    )(x, W, hist)

    part = _edge_scatter_kernel(npad, nh, cpt)(g2, src_sc, dst_sc, zhalf)

    fb = 400
    assert n % fb == 0

    def fin_body(p_ref, g_ref, dinv_ref, b_ref, o_ref):
        dinv = dinv_ref[...]
        a0 = (p_ref[0] + g_ref[0]) * dinv
        a1 = (p_ref[1] + g_ref[1]) * dinv
        acc = jnp.concatenate([a0, a1], axis=1)
        o_ref[...] = jnp.maximum(acc + b_ref[...], 0.0)

    out = pl.pallas_call(
        fin_body,
        grid=(n // fb,),
        in_specs=[
            pl.BlockSpec((_NC, fb, nh), lambda i: (0, i, 0)),
            pl.BlockSpec((_NC, fb, nh), lambda i: (0, i, 0)),
            pl.BlockSpec((fb, 1), lambda i: (i, 0)),
            pl.BlockSpec((1, nhid), lambda i: (0, 0)),
        ],
        out_specs=pl.BlockSpec((fb, nhid), lambda i: (i, 0)),
        out_shape=jax.ShapeDtypeStruct((n, nhid), jnp.float32),
    )(part, g2, dinv2, b.reshape(1, nhid))

    return out
